# baseline (speedup 1.0000x reference)
import jax
import jax.numpy as jnp
from jax.experimental import pallas as pl
from jax.experimental.pallas import tpu as pltpu

NUM_PATCHES = 576
LATENT_DIM = 768
BATCH = 64

BB = 4
NBUF = 6


def _outer(x_hbm, pos_ref, out_hbm):
    def body(x_blk, out_blk):
        out_blk[...] = x_blk[...] + pos_ref[...]

    pipeline = pltpu.emit_pipeline(
        body,
        grid=(BATCH // BB,),
        in_specs=[
            pl.BlockSpec((BB, NUM_PATCHES, LATENT_DIM), lambda b: (b, 0, 0),
                         pipeline_mode=pl.Buffered(buffer_count=NBUF)),
        ],
        out_specs=[
            pl.BlockSpec((BB, NUM_PATCHES, LATENT_DIM), lambda b: (b, 0, 0)),
        ],
    )
    pipeline(x_hbm, out_hbm)


def kernel(x, pos_table):
    return pl.pallas_call(
        _outer,
        in_specs=[
            pl.BlockSpec(memory_space=pltpu.HBM),
            pl.BlockSpec(memory_space=pltpu.VMEM),
        ],
        out_specs=pl.BlockSpec(memory_space=pltpu.HBM),
        out_shape=jax.ShapeDtypeStruct((BATCH, NUM_PATCHES, LATENT_DIM), x.dtype),
    )(x, pos_table)


# speedup vs baseline: 1.0207x; 1.0207x over previous
import jax
import jax.numpy as jnp
from jax.experimental import pallas as pl

NUM_PATCHES = 576
LATENT_DIM = 768
BATCH = 64

BB = 8


def _add_kernel(x_ref, pos_ref, out_ref):
    out_ref[...] = x_ref[...] + pos_ref[...]


def kernel(x, pos_table):
    return pl.pallas_call(
        _add_kernel,
        grid=(BATCH // BB,),
        in_specs=[
            pl.BlockSpec((BB, NUM_PATCHES, LATENT_DIM), lambda b: (b, 0, 0)),
            pl.BlockSpec((NUM_PATCHES, LATENT_DIM), lambda b: (0, 0)),
        ],
        out_specs=pl.BlockSpec((BB, NUM_PATCHES, LATENT_DIM), lambda b: (b, 0, 0)),
        out_shape=jax.ShapeDtypeStruct((BATCH, NUM_PATCHES, LATENT_DIM), x.dtype),
    )(x, pos_table)
